# baseline (speedup 1.0000x reference)
import functools

import jax
import jax.numpy as jnp
from jax import lax
from jax.experimental import pallas as pl
from jax.experimental.pallas import tpu as pltpu
from jax.experimental.pallas import tpu_sc as plsc

NUM_CODES = 8192
CODE_DIM = 64
COMMITMENT_COST = 0.25
N_TOK = 16 * 576

TT = 512
TK = 2048
_NKS = NUM_CODES // TK
_NACC = 4


_GRP = 256


def _argmin_kernel(w2_ref, z_ref, a_ref, c_ref, idx_ref, dsum_ref,
                   val_scr, pay_scr):
    k = pl.program_id(0)
    i = pl.program_id(1)
    z = z_ref[...]
    a = a_ref[...]
    c = c_ref[...]
    tds = pl.ds(i * TT, TT)

    @pl.when(k == 0)
    def _init():
        val_scr[:, tds] = jnp.full((8 * _NACC, TT), jnp.inf, jnp.float32)
        pay_scr[:, tds] = jnp.zeros((8 * _NACC, TT), jnp.float32)

    vals = [val_scr[8 * t:8 * (t + 1), tds] for t in range(_NACC)]
    pays = [pay_scr[8 * t:8 * (t + 1), tds] for t in range(_NACC)]
    kbase = (k * (TK // 8)).astype(jnp.float32)
    for j in range(TK // _GRP):
        m2j = lax.dot_general(
            w2_ref[j * _GRP:(j + 1) * _GRP, :], z, (((1,), (1,)), ((), ())),
            preferred_element_type=jnp.float32,
        )
        for gg in range(_GRP // 8):
            g = j * (_GRP // 8) + gg
            d = (a - m2j[gg * 8:(gg + 1) * 8, :]) + c[g * 8:(g + 1) * 8, :]
            pg = kbase + float(g)
            t = g % _NACC
            better = d < vals[t]
            vals[t] = jnp.minimum(d, vals[t])
            pays[t] = jnp.where(better, pg, pays[t])

    @pl.when(k < _NKS - 1)
    def _carry():
        val_scr[:, tds] = jnp.concatenate(vals, axis=0)
        pay_scr[:, tds] = jnp.concatenate(pays, axis=0)

    @pl.when(k == _NKS - 1)
    def _finish():
        sub_iota = lax.broadcasted_iota(jnp.int32, (8, 1), 0).astype(jnp.float32)
        mval, midx = vals[0], pays[0] * 8.0 + sub_iota
        for t in range(1, _NACC):
            v2, i2 = vals[t], pays[t] * 8.0 + sub_iota
            b = (v2 < mval) | ((v2 == mval) & (i2 < midx))
            mval = jnp.where(b, v2, mval)
            midx = jnp.where(b, i2, midx)
        mcol = jnp.min(mval, axis=0, keepdims=True)
        sel = jnp.where(mval == mcol, midx, float(NUM_CODES))
        icol = jnp.min(sel, axis=0, keepdims=True)
        idx_ref[...] = icol.astype(jnp.int32)
        part = jnp.sum(mcol)
        prev = jnp.where(i == 0, 0.0, dsum_ref[0, 0])
        dsum_ref[0, 0] = prev + part


def _build_argmin(interpret: bool = False):
    return pl.pallas_call(
        _argmin_kernel,
        grid=(_NKS, N_TOK // TT),
        in_specs=[
            pl.BlockSpec((TK, CODE_DIM), lambda k, i: (k, 0)),
            pl.BlockSpec((TT, CODE_DIM), lambda k, i: (i, 0)),
            pl.BlockSpec((1, TT), lambda k, i: (0, i)),
            pl.BlockSpec((TK, 1), lambda k, i: (k, 0)),
        ],
        out_specs=[
            pl.BlockSpec((1, TT), lambda k, i: (0, i)),
            pl.BlockSpec((1, 1), lambda k, i: (0, 0),
                         memory_space=pltpu.SMEM),
        ],
        out_shape=[
            jax.ShapeDtypeStruct((1, N_TOK), jnp.int32),
            jax.ShapeDtypeStruct((1, 1), jnp.float32),
        ],
        scratch_shapes=[
            pltpu.VMEM((8 * _NACC, N_TOK), jnp.float32),
            pltpu.VMEM((8 * _NACC, N_TOK), jnp.float32),
        ],
        compiler_params=pltpu.CompilerParams(
            dimension_semantics=("arbitrary", "arbitrary"),
        ),
        interpret=interpret,
    )


_NC = 2
_NS = 16
_NW = _NC * _NS
_TPW = N_TOK // _NW
_CHUNK = 96
_NCHUNK = _TPW // _CHUNK


def _sc_gather_body(w2_hbm, idx_hbm, z_hbm, out_hbm,
                    idx_v, rows_v, z_v, out_v, sem):
    wid = lax.axis_index("s") * _NC + lax.axis_index("c")
    base = wid * _TPW
    pltpu.sync_copy(idx_hbm.at[pl.ds(base, _TPW)], idx_v)
    cps = [
        pltpu.async_copy(
            w2_hbm.at[idx_v.at[pl.ds(c * _CHUNK, _CHUNK)]],
            rows_v.at[pl.ds(c * _CHUNK, _CHUNK)],
            sem,
        )
        for c in range(_NCHUNK)
    ]
    pltpu.sync_copy(z_hbm.at[pl.ds(base, _TPW)], z_v)
    for cp in cps:
        cp.wait()

    def body(r, carry):
        for j in range(CODE_DIM // 16):
            sl = pl.ds(j * 16, 16)
            q = rows_v[r, sl]
            zz = z_v[r, sl]
            out_v[r, sl] = zz + (q - zz)
        return carry

    lax.fori_loop(0, _TPW, body, 0)
    pltpu.sync_copy(out_v, out_hbm.at[pl.ds(base, _TPW)])


@functools.lru_cache(maxsize=1)
def _build_sc_gather():
    return pl.kernel(
        _sc_gather_body,
        mesh=plsc.VectorSubcoreMesh(core_axis_name="c", subcore_axis_name="s"),
        out_type=jax.ShapeDtypeStruct((N_TOK, CODE_DIM), jnp.float32),
        scratch_types=[
            pltpu.VMEM((_TPW,), jnp.int32),
            pltpu.VMEM((_TPW, 2 * CODE_DIM), jnp.float32),
            pltpu.VMEM((_TPW, CODE_DIM), jnp.float32),
            pltpu.VMEM((_TPW, CODE_DIM), jnp.float32),
            pltpu.SemaphoreType.DMA,
        ],
    )


def kernel(z, W):
    flat_z = z.reshape(-1, CODE_DIM)
    a = jnp.sum(flat_z ** 2, axis=1, keepdims=True).reshape(1, N_TOK)
    c = jnp.sum(W ** 2, axis=1).reshape(NUM_CODES, 1)
    w2 = W + W
    idx2, dsum = _build_argmin()(w2, flat_z, a, c)
    idx_flat = idx2[0, :]
    w_wide = jnp.concatenate([W, W], axis=1)
    z_q_ste = _build_sc_gather()(w_wide, idx_flat, flat_z).reshape(z.shape)
    vq_loss = COMMITMENT_COST * (dsum[0, 0] / (N_TOK * CODE_DIM))
    return (z_q_ste, idx_flat.reshape(z.shape[:2]), vq_loss)


# speedup vs baseline: 1.0406x; 1.0406x over previous
import functools

import jax
import jax.numpy as jnp
from jax import lax
from jax.experimental import pallas as pl
from jax.experimental.pallas import tpu as pltpu
from jax.experimental.pallas import tpu_sc as plsc

NUM_CODES = 8192
CODE_DIM = 64
COMMITMENT_COST = 0.25
N_TOK = 16 * 576

TT = 512
TK = 2048
_NKS = NUM_CODES // TK
_NACC = 4


_GRP = 256


_NI = N_TOK // TT
_LSCALE = COMMITMENT_COST / (N_TOK * CODE_DIM)


def _argmin_kernel(w_ref, z2_ref, a_ref, idx_ref, dsum_ref, wwide_ref,
                   val_scr, pay_scr, c_scr):
    k = pl.program_id(0)
    i = pl.program_id(1)

    @pl.when(i == 0)
    def _per_code_tile():
        w_t = w_ref[...]
        c_scr[...] = jnp.sum(w_t * w_t, axis=1, keepdims=True)
        wwide_ref[:, 0:CODE_DIM] = w_t
        wwide_ref[:, CODE_DIM:2 * CODE_DIM] = w_t

    z = z2_ref[...]
    a = a_ref[...]
    c = c_scr[...]
    tds = pl.ds(i * TT, TT)

    @pl.when(k == 0)
    def _init():
        val_scr[:, tds] = jnp.full((8 * _NACC, TT), jnp.inf, jnp.float32)
        pay_scr[:, tds] = jnp.zeros((8 * _NACC, TT), jnp.float32)

    vals = [val_scr[8 * t:8 * (t + 1), tds] for t in range(_NACC)]
    pays = [pay_scr[8 * t:8 * (t + 1), tds] for t in range(_NACC)]
    kbase = (k * (TK // 8)).astype(jnp.float32)
    for j in range(TK // _GRP):
        m2j = lax.dot_general(
            w_ref[j * _GRP:(j + 1) * _GRP, :], z, (((1,), (1,)), ((), ())),
            preferred_element_type=jnp.float32,
        )
        for gg in range(_GRP // 8):
            g = j * (_GRP // 8) + gg
            d = (a - m2j[gg * 8:(gg + 1) * 8, :]) + c[g * 8:(g + 1) * 8, :]
            pg = kbase + float(g)
            t = g % _NACC
            better = d < vals[t]
            vals[t] = jnp.minimum(d, vals[t])
            pays[t] = jnp.where(better, pg, pays[t])

    @pl.when(k < _NKS - 1)
    def _carry():
        val_scr[:, tds] = jnp.concatenate(vals, axis=0)
        pay_scr[:, tds] = jnp.concatenate(pays, axis=0)

    @pl.when(k == _NKS - 1)
    def _finish():
        sub_iota = lax.broadcasted_iota(jnp.int32, (8, 1), 0).astype(jnp.float32)
        mval, midx = vals[0], pays[0] * 8.0 + sub_iota
        for t in range(1, _NACC):
            v2, i2 = vals[t], pays[t] * 8.0 + sub_iota
            b = (v2 < mval) | ((v2 == mval) & (i2 < midx))
            mval = jnp.where(b, v2, mval)
            midx = jnp.where(b, i2, midx)
        mcol = jnp.min(mval, axis=0, keepdims=True)
        sel = jnp.where(mval == mcol, midx, float(NUM_CODES))
        icol = jnp.min(sel, axis=0, keepdims=True)
        idx_ref[...] = icol.astype(jnp.int32)
        part = jnp.sum(mcol)
        prev = jnp.where(i == 0, 0.0, dsum_ref[0, 0])
        total = prev + part
        dsum_ref[0, 0] = jnp.where(i == _NI - 1, total * _LSCALE, total)


def _build_argmin(interpret: bool = False):
    return pl.pallas_call(
        _argmin_kernel,
        grid=(_NKS, N_TOK // TT),
        in_specs=[
            pl.BlockSpec((TK, CODE_DIM), lambda k, i: (k, 0)),
            pl.BlockSpec((TT, CODE_DIM), lambda k, i: (i, 0)),
            pl.BlockSpec((1, TT), lambda k, i: (0, i)),
        ],
        out_specs=[
            pl.BlockSpec((1, TT), lambda k, i: (0, i)),
            pl.BlockSpec((1, 1), lambda k, i: (0, 0),
                         memory_space=pltpu.SMEM),
            pl.BlockSpec((TK, 2 * CODE_DIM), lambda k, i: (k, 0)),
        ],
        out_shape=[
            jax.ShapeDtypeStruct((1, N_TOK), jnp.int32),
            jax.ShapeDtypeStruct((1, 1), jnp.float32),
            jax.ShapeDtypeStruct((NUM_CODES, 2 * CODE_DIM), jnp.float32),
        ],
        scratch_shapes=[
            pltpu.VMEM((8 * _NACC, N_TOK), jnp.float32),
            pltpu.VMEM((8 * _NACC, N_TOK), jnp.float32),
            pltpu.VMEM((TK, 1), jnp.float32),
        ],
        compiler_params=pltpu.CompilerParams(
            dimension_semantics=("arbitrary", "arbitrary"),
        ),
        interpret=interpret,
    )


_NC = 2
_NS = 16
_NW = _NC * _NS
_TPW = N_TOK // _NW
_CHUNK = 96
_NCHUNK = _TPW // _CHUNK


def _sc_gather_body(w2_hbm, idx_hbm, z_hbm, out_hbm,
                    idx_v, rows_v, z_v, out_v, sem):
    wid = lax.axis_index("s") * _NC + lax.axis_index("c")
    base = wid * _TPW
    pltpu.sync_copy(idx_hbm.at[pl.ds(base, _TPW)], idx_v)
    cps = [
        pltpu.async_copy(
            w2_hbm.at[idx_v.at[pl.ds(c * _CHUNK, _CHUNK)]],
            rows_v.at[pl.ds(c * _CHUNK, _CHUNK)],
            sem,
        )
        for c in range(_NCHUNK)
    ]
    pltpu.sync_copy(z_hbm.at[pl.ds(base, _TPW)], z_v)
    for cp in cps:
        cp.wait()

    def body(r, carry):
        for j in range(CODE_DIM // 16):
            sl = pl.ds(j * 16, 16)
            q = rows_v[r, sl]
            zz = z_v[r, sl]
            out_v[r, sl] = zz + (q - zz)
        return carry

    lax.fori_loop(0, _TPW, body, 0)
    pltpu.sync_copy(out_v, out_hbm.at[pl.ds(base, _TPW)])


@functools.lru_cache(maxsize=1)
def _build_sc_gather():
    return pl.kernel(
        _sc_gather_body,
        mesh=plsc.VectorSubcoreMesh(core_axis_name="c", subcore_axis_name="s"),
        out_type=jax.ShapeDtypeStruct((N_TOK, CODE_DIM), jnp.float32),
        scratch_types=[
            pltpu.VMEM((_TPW,), jnp.int32),
            pltpu.VMEM((_TPW, 2 * CODE_DIM), jnp.float32),
            pltpu.VMEM((_TPW, CODE_DIM), jnp.float32),
            pltpu.VMEM((_TPW, CODE_DIM), jnp.float32),
            pltpu.SemaphoreType.DMA,
        ],
    )


def kernel(z, W):
    flat_z = z.reshape(-1, CODE_DIM)
    a = jnp.sum(flat_z ** 2, axis=1, keepdims=True).reshape(1, N_TOK)
    z2 = flat_z + flat_z
    idx2, dsum, w_wide = _build_argmin()(W, z2, a)
    idx_flat = idx2[0, :]
    z_q_ste = _build_sc_gather()(w_wide, idx_flat, flat_z).reshape(z.shape)
    return (z_q_ste, idx_flat.reshape(z.shape[:2]), dsum[0, 0])


# speedup vs baseline: 1.2636x; 1.2143x over previous
import functools

import jax
import jax.numpy as jnp
from jax import lax
from jax.experimental import pallas as pl
from jax.experimental.pallas import tpu as pltpu
from jax.experimental.pallas import tpu_sc as plsc

NUM_CODES = 8192
CODE_DIM = 64
COMMITMENT_COST = 0.25
N_TOK = 16 * 576

TT = 512
TK = 8192
_NKS = NUM_CODES // TK
_NACC = 4


_GRP = 256


_NI = N_TOK // TT
_LSCALE = COMMITMENT_COST / (N_TOK * CODE_DIM)


def _argmin_kernel(w_ref, z2_ref, a_ref, idx_ref, dsum_ref, wwide_ref,
                   c_scr):
    i = pl.program_id(0)

    @pl.when(i == 0)
    def _per_code_tile():
        w_t = w_ref[...]
        c_scr[...] = jnp.sum(w_t * w_t, axis=1, keepdims=True)
        wwide_ref[:, 0:CODE_DIM] = w_t
        wwide_ref[:, CODE_DIM:2 * CODE_DIM] = w_t

    z = z2_ref[...]
    a = a_ref[...]
    c = c_scr[...]

    vals = [jnp.full((8, TT), jnp.inf, jnp.float32) for _ in range(_NACC)]
    pays = [jnp.zeros((8, TT), jnp.float32) for _ in range(_NACC)]
    for j in range(TK // _GRP):
        m2j = lax.dot_general(
            w_ref[j * _GRP:(j + 1) * _GRP, :], z, (((1,), (1,)), ((), ())),
            preferred_element_type=jnp.float32,
        )
        for gg in range(_GRP // 8):
            g = j * (_GRP // 8) + gg
            d = (a - m2j[gg * 8:(gg + 1) * 8, :]) + c[g * 8:(g + 1) * 8, :]
            pg = float(g)
            t = g % _NACC
            better = d < vals[t]
            vals[t] = jnp.minimum(d, vals[t])
            pays[t] = jnp.where(better, pg, pays[t])

    sub_iota = lax.broadcasted_iota(jnp.int32, (8, 1), 0).astype(jnp.float32)
    mval, midx = vals[0], pays[0] * 8.0 + sub_iota
    for t in range(1, _NACC):
        v2, i2 = vals[t], pays[t] * 8.0 + sub_iota
        b = (v2 < mval) | ((v2 == mval) & (i2 < midx))
        mval = jnp.where(b, v2, mval)
        midx = jnp.where(b, i2, midx)
    mcol = jnp.min(mval, axis=0, keepdims=True)
    sel = jnp.where(mval == mcol, midx, float(NUM_CODES))
    icol = jnp.min(sel, axis=0, keepdims=True)
    idx_ref[...] = icol.astype(jnp.int32)
    part = jnp.sum(mcol)
    prev = jnp.where(i == 0, 0.0, dsum_ref[0, 0])
    total = prev + part
    dsum_ref[0, 0] = jnp.where(i == _NI - 1, total * _LSCALE, total)


def _build_argmin(interpret: bool = False):
    return pl.pallas_call(
        _argmin_kernel,
        grid=(N_TOK // TT,),
        in_specs=[
            pl.BlockSpec((TK, CODE_DIM), lambda i: (0, 0)),
            pl.BlockSpec((TT, CODE_DIM), lambda i: (i, 0)),
            pl.BlockSpec((1, TT), lambda i: (0, i)),
        ],
        out_specs=[
            pl.BlockSpec((1, TT), lambda i: (0, i)),
            pl.BlockSpec((1, 1), lambda i: (0, 0),
                         memory_space=pltpu.SMEM),
            pl.BlockSpec((TK, 2 * CODE_DIM), lambda i: (0, 0)),
        ],
        out_shape=[
            jax.ShapeDtypeStruct((1, N_TOK), jnp.int32),
            jax.ShapeDtypeStruct((1, 1), jnp.float32),
            jax.ShapeDtypeStruct((NUM_CODES, 2 * CODE_DIM), jnp.float32),
        ],
        scratch_shapes=[
            pltpu.VMEM((TK, 1), jnp.float32),
        ],
        compiler_params=pltpu.CompilerParams(
            dimension_semantics=("arbitrary",),
        ),
        interpret=interpret,
    )


_NC = 2
_NS = 16
_NW = _NC * _NS
_TPW = N_TOK // _NW
_CHUNK = 96
_NCHUNK = _TPW // _CHUNK


def _sc_gather_body(w2_hbm, idx_hbm, z_hbm, out_hbm,
                    idx_v, rows_v, z_v, out_v, sem):
    wid = lax.axis_index("s") * _NC + lax.axis_index("c")
    base = wid * _TPW
    pltpu.sync_copy(idx_hbm.at[pl.ds(base, _TPW)], idx_v)
    cps = [
        pltpu.async_copy(
            w2_hbm.at[idx_v.at[pl.ds(c * _CHUNK, _CHUNK)]],
            rows_v.at[pl.ds(c * _CHUNK, _CHUNK)],
            sem,
        )
        for c in range(_NCHUNK)
    ]
    pltpu.sync_copy(z_hbm.at[pl.ds(base, _TPW)], z_v)
    for cp in cps:
        cp.wait()

    def body(r, carry):
        for j in range(CODE_DIM // 16):
            sl = pl.ds(j * 16, 16)
            q = rows_v[r, sl]
            zz = z_v[r, sl]
            out_v[r, sl] = zz + (q - zz)
        return carry

    lax.fori_loop(0, _TPW, body, 0)
    pltpu.sync_copy(out_v, out_hbm.at[pl.ds(base, _TPW)])


@functools.lru_cache(maxsize=1)
def _build_sc_gather():
    return pl.kernel(
        _sc_gather_body,
        mesh=plsc.VectorSubcoreMesh(core_axis_name="c", subcore_axis_name="s"),
        out_type=jax.ShapeDtypeStruct((N_TOK, CODE_DIM), jnp.float32),
        scratch_types=[
            pltpu.VMEM((_TPW,), jnp.int32),
            pltpu.VMEM((_TPW, 2 * CODE_DIM), jnp.float32),
            pltpu.VMEM((_TPW, CODE_DIM), jnp.float32),
            pltpu.VMEM((_TPW, CODE_DIM), jnp.float32),
            pltpu.SemaphoreType.DMA,
        ],
    )


def kernel(z, W):
    flat_z = z.reshape(-1, CODE_DIM)
    a = jnp.sum(flat_z ** 2, axis=1, keepdims=True).reshape(1, N_TOK)
    z2 = flat_z + flat_z
    idx2, dsum, w_wide = _build_argmin()(W, z2, a)
    idx_flat = idx2[0, :]
    z_q_ste = _build_sc_gather()(w_wide, idx_flat, flat_z).reshape(z.shape)
    return (z_q_ste, idx_flat.reshape(z.shape[:2]), dsum[0, 0])


# speedup vs baseline: 1.2656x; 1.0016x over previous
import functools

import jax
import jax.numpy as jnp
from jax import lax
from jax.experimental import pallas as pl
from jax.experimental.pallas import tpu as pltpu
from jax.experimental.pallas import tpu_sc as plsc

NUM_CODES = 8192
CODE_DIM = 64
COMMITMENT_COST = 0.25
N_TOK = 16 * 576

TT = 512
TK = 8192
_NKS = NUM_CODES // TK
_NACC = 2


_GRP = 256


_NI = N_TOK // TT
_LSCALE = COMMITMENT_COST / (N_TOK * CODE_DIM)


def _argmin_kernel(w_ref, z2_ref, a_ref, idx_ref, dsum_ref, wwide_ref,
                   c_scr):
    i = pl.program_id(0)

    @pl.when(i == 0)
    def _per_code_tile():
        w_t = w_ref[...]
        c_scr[...] = jnp.sum(w_t * w_t, axis=1, keepdims=True)
        wwide_ref[:, 0:CODE_DIM] = w_t
        wwide_ref[:, CODE_DIM:2 * CODE_DIM] = w_t

    z = z2_ref[...]
    a = a_ref[...]
    c = c_scr[...]

    vals = [jnp.full((8, TT), jnp.inf, jnp.float32) for _ in range(_NACC)]
    pays = [jnp.zeros((8, TT), jnp.float32) for _ in range(_NACC)]
    for j in range(TK // _GRP):
        m2j = lax.dot_general(
            w_ref[j * _GRP:(j + 1) * _GRP, :], z, (((1,), (1,)), ((), ())),
            preferred_element_type=jnp.float32,
        )
        for gg in range(_GRP // 8):
            g = j * (_GRP // 8) + gg
            d = (a - m2j[gg * 8:(gg + 1) * 8, :]) + c[g * 8:(g + 1) * 8, :]
            pg = float(g)
            t = g % _NACC
            better = d < vals[t]
            vals[t] = jnp.minimum(d, vals[t])
            pays[t] = jnp.where(better, pg, pays[t])

    sub_iota = lax.broadcasted_iota(jnp.int32, (8, 1), 0).astype(jnp.float32)
    mval, midx = vals[0], pays[0] * 8.0 + sub_iota
    for t in range(1, _NACC):
        v2, i2 = vals[t], pays[t] * 8.0 + sub_iota
        b = (v2 < mval) | ((v2 == mval) & (i2 < midx))
        mval = jnp.where(b, v2, mval)
        midx = jnp.where(b, i2, midx)
    mcol = jnp.min(mval, axis=0, keepdims=True)
    sel = jnp.where(mval == mcol, midx, float(NUM_CODES))
    icol = jnp.min(sel, axis=0, keepdims=True)
    idx_ref[...] = icol.astype(jnp.int32)
    part = jnp.sum(mcol)
    prev = jnp.where(i == 0, 0.0, dsum_ref[0, 0])
    total = prev + part
    dsum_ref[0, 0] = jnp.where(i == _NI - 1, total * _LSCALE, total)


def _build_argmin(interpret: bool = False):
    return pl.pallas_call(
        _argmin_kernel,
        grid=(N_TOK // TT,),
        in_specs=[
            pl.BlockSpec((TK, CODE_DIM), lambda i: (0, 0)),
            pl.BlockSpec((TT, CODE_DIM), lambda i: (i, 0)),
            pl.BlockSpec((1, TT), lambda i: (0, i)),
        ],
        out_specs=[
            pl.BlockSpec((1, TT), lambda i: (0, i)),
            pl.BlockSpec((1, 1), lambda i: (0, 0),
                         memory_space=pltpu.SMEM),
            pl.BlockSpec((TK, 2 * CODE_DIM), lambda i: (0, 0)),
        ],
        out_shape=[
            jax.ShapeDtypeStruct((1, N_TOK), jnp.int32),
            jax.ShapeDtypeStruct((1, 1), jnp.float32),
            jax.ShapeDtypeStruct((NUM_CODES, 2 * CODE_DIM), jnp.float32),
        ],
        scratch_shapes=[
            pltpu.VMEM((TK, 1), jnp.float32),
        ],
        compiler_params=pltpu.CompilerParams(
            dimension_semantics=("arbitrary",),
        ),
        interpret=interpret,
    )


_NC = 2
_NS = 16
_NW = _NC * _NS
_TPW = N_TOK // _NW
_CHUNK = 96
_NCHUNK = _TPW // _CHUNK


def _sc_gather_body(w2_hbm, idx_hbm, z_hbm, out_hbm,
                    idx_v, rows_v, z_v, out_v, sem):
    wid = lax.axis_index("s") * _NC + lax.axis_index("c")
    base = wid * _TPW
    pltpu.sync_copy(idx_hbm.at[pl.ds(base, _TPW)], idx_v)
    cps = [
        pltpu.async_copy(
            w2_hbm.at[idx_v.at[pl.ds(c * _CHUNK, _CHUNK)]],
            rows_v.at[pl.ds(c * _CHUNK, _CHUNK)],
            sem,
        )
        for c in range(_NCHUNK)
    ]
    pltpu.sync_copy(z_hbm.at[pl.ds(base, _TPW)], z_v)
    for cp in cps:
        cp.wait()

    def body(r, carry):
        for j in range(CODE_DIM // 16):
            sl = pl.ds(j * 16, 16)
            q = rows_v[r, sl]
            zz = z_v[r, sl]
            out_v[r, sl] = zz + (q - zz)
        return carry

    lax.fori_loop(0, _TPW, body, 0)
    pltpu.sync_copy(out_v, out_hbm.at[pl.ds(base, _TPW)])


@functools.lru_cache(maxsize=1)
def _build_sc_gather():
    return pl.kernel(
        _sc_gather_body,
        mesh=plsc.VectorSubcoreMesh(core_axis_name="c", subcore_axis_name="s"),
        out_type=jax.ShapeDtypeStruct((N_TOK, CODE_DIM), jnp.float32),
        scratch_types=[
            pltpu.VMEM((_TPW,), jnp.int32),
            pltpu.VMEM((_TPW, 2 * CODE_DIM), jnp.float32),
            pltpu.VMEM((_TPW, CODE_DIM), jnp.float32),
            pltpu.VMEM((_TPW, CODE_DIM), jnp.float32),
            pltpu.SemaphoreType.DMA,
        ],
    )


def kernel(z, W):
    flat_z = z.reshape(-1, CODE_DIM)
    a = jnp.sum(flat_z ** 2, axis=1, keepdims=True).reshape(1, N_TOK)
    z2 = flat_z + flat_z
    idx2, dsum, w_wide = _build_argmin()(W, z2, a)
    idx_flat = idx2[0, :]
    z_q_ste = _build_sc_gather()(w_wide, idx_flat, flat_z).reshape(z.shape)
    return (z_q_ste, idx_flat.reshape(z.shape[:2]), dsum[0, 0])


# speedup vs baseline: 1.2817x; 1.0127x over previous
import functools

import jax
import jax.numpy as jnp
from jax import lax
from jax.experimental import pallas as pl
from jax.experimental.pallas import tpu as pltpu
from jax.experimental.pallas import tpu_sc as plsc

NUM_CODES = 8192
CODE_DIM = 64
COMMITMENT_COST = 0.25
N_TOK = 16 * 576

TT = 1024
TK = 8192
_NKS = NUM_CODES // TK
_NACC = 2


_GRP = 256


_NI = N_TOK // TT
_LSCALE = COMMITMENT_COST / (N_TOK * CODE_DIM)


def _argmin_kernel(w_ref, z2_ref, a_ref, idx_ref, dsum_ref, wwide_ref,
                   c_scr):
    i = pl.program_id(0)

    @pl.when(i == 0)
    def _per_code_tile():
        w_t = w_ref[...]
        c_scr[...] = jnp.sum(w_t * w_t, axis=1, keepdims=True)
        wwide_ref[:, 0:CODE_DIM] = w_t
        wwide_ref[:, CODE_DIM:2 * CODE_DIM] = w_t

    z = z2_ref[...]
    a = a_ref[...]
    c = c_scr[...]

    vals = [jnp.full((8, TT), jnp.inf, jnp.float32) for _ in range(_NACC)]
    pays = [jnp.zeros((8, TT), jnp.float32) for _ in range(_NACC)]
    for j in range(TK // _GRP):
        m2j = lax.dot_general(
            w_ref[j * _GRP:(j + 1) * _GRP, :], z, (((1,), (1,)), ((), ())),
            preferred_element_type=jnp.float32,
        )
        for gg in range(_GRP // 8):
            g = j * (_GRP // 8) + gg
            d = (a - m2j[gg * 8:(gg + 1) * 8, :]) + c[g * 8:(g + 1) * 8, :]
            pg = float(g)
            t = g % _NACC
            better = d < vals[t]
            vals[t] = jnp.minimum(d, vals[t])
            pays[t] = jnp.where(better, pg, pays[t])

    sub_iota = lax.broadcasted_iota(jnp.int32, (8, 1), 0).astype(jnp.float32)
    mval, midx = vals[0], pays[0] * 8.0 + sub_iota
    for t in range(1, _NACC):
        v2, i2 = vals[t], pays[t] * 8.0 + sub_iota
        b = (v2 < mval) | ((v2 == mval) & (i2 < midx))
        mval = jnp.where(b, v2, mval)
        midx = jnp.where(b, i2, midx)
    mcol = jnp.min(mval, axis=0, keepdims=True)
    sel = jnp.where(mval == mcol, midx, float(NUM_CODES))
    icol = jnp.min(sel, axis=0, keepdims=True)
    idx_ref[...] = icol.astype(jnp.int32)
    part = jnp.sum(mcol)
    prev = jnp.where(i == 0, 0.0, dsum_ref[0, 0])
    total = prev + part
    dsum_ref[0, 0] = jnp.where(i == _NI - 1, total * _LSCALE, total)


def _build_argmin(interpret: bool = False):
    return pl.pallas_call(
        _argmin_kernel,
        grid=(N_TOK // TT,),
        in_specs=[
            pl.BlockSpec((TK, CODE_DIM), lambda i: (0, 0)),
            pl.BlockSpec((TT, CODE_DIM), lambda i: (i, 0)),
            pl.BlockSpec((1, TT), lambda i: (0, i)),
        ],
        out_specs=[
            pl.BlockSpec((1, TT), lambda i: (0, i)),
            pl.BlockSpec((1, 1), lambda i: (0, 0),
                         memory_space=pltpu.SMEM),
            pl.BlockSpec((TK, 2 * CODE_DIM), lambda i: (0, 0)),
        ],
        out_shape=[
            jax.ShapeDtypeStruct((1, N_TOK), jnp.int32),
            jax.ShapeDtypeStruct((1, 1), jnp.float32),
            jax.ShapeDtypeStruct((NUM_CODES, 2 * CODE_DIM), jnp.float32),
        ],
        scratch_shapes=[
            pltpu.VMEM((TK, 1), jnp.float32),
        ],
        compiler_params=pltpu.CompilerParams(
            dimension_semantics=("arbitrary",),
        ),
        interpret=interpret,
    )


_NC = 2
_NS = 16
_NW = _NC * _NS
_TPW = N_TOK // _NW
_CHUNK = 96
_NCHUNK = _TPW // _CHUNK


def _sc_gather_body(w2_hbm, idx_hbm, z_hbm, out_hbm,
                    idx_v, rows_v, z_v, out_v, sem):
    wid = lax.axis_index("s") * _NC + lax.axis_index("c")
    base = wid * _TPW
    pltpu.sync_copy(idx_hbm.at[pl.ds(base, _TPW)], idx_v)
    cps = [
        pltpu.async_copy(
            w2_hbm.at[idx_v.at[pl.ds(c * _CHUNK, _CHUNK)]],
            rows_v.at[pl.ds(c * _CHUNK, _CHUNK)],
            sem,
        )
        for c in range(_NCHUNK)
    ]
    pltpu.sync_copy(z_hbm.at[pl.ds(base, _TPW)], z_v)
    for cp in cps:
        cp.wait()

    def body(r, carry):
        for j in range(CODE_DIM // 16):
            sl = pl.ds(j * 16, 16)
            q = rows_v[r, sl]
            zz = z_v[r, sl]
            out_v[r, sl] = zz + (q - zz)
        return carry

    lax.fori_loop(0, _TPW, body, 0)
    pltpu.sync_copy(out_v, out_hbm.at[pl.ds(base, _TPW)])


@functools.lru_cache(maxsize=1)
def _build_sc_gather():
    return pl.kernel(
        _sc_gather_body,
        mesh=plsc.VectorSubcoreMesh(core_axis_name="c", subcore_axis_name="s"),
        out_type=jax.ShapeDtypeStruct((N_TOK, CODE_DIM), jnp.float32),
        scratch_types=[
            pltpu.VMEM((_TPW,), jnp.int32),
            pltpu.VMEM((_TPW, 2 * CODE_DIM), jnp.float32),
            pltpu.VMEM((_TPW, CODE_DIM), jnp.float32),
            pltpu.VMEM((_TPW, CODE_DIM), jnp.float32),
            pltpu.SemaphoreType.DMA,
        ],
    )


def kernel(z, W):
    flat_z = z.reshape(-1, CODE_DIM)
    a = jnp.sum(flat_z ** 2, axis=1, keepdims=True).reshape(1, N_TOK)
    z2 = flat_z + flat_z
    idx2, dsum, w_wide = _build_argmin()(W, z2, a)
    idx_flat = idx2[0, :]
    z_q_ste = _build_sc_gather()(w_wide, idx_flat, flat_z).reshape(z.shape)
    return (z_q_ste, idx_flat.reshape(z.shape[:2]), dsum[0, 0])


# speedup vs baseline: 1.3022x; 1.0161x over previous
import functools

import jax
import jax.numpy as jnp
from jax import lax
from jax.experimental import pallas as pl
from jax.experimental.pallas import tpu as pltpu
from jax.experimental.pallas import tpu_sc as plsc

NUM_CODES = 8192
CODE_DIM = 64
COMMITMENT_COST = 0.25
N_TOK = 16 * 576

TT = 1024
TK = 8192
_NKS = NUM_CODES // TK
_NACC = 2


_GRP = 512


_NI = N_TOK // TT
_LSCALE = COMMITMENT_COST / (N_TOK * CODE_DIM)


def _argmin_kernel(w_ref, z2_ref, a_ref, idx_ref, dsum_ref, wwide_ref,
                   c_scr):
    i = pl.program_id(0)

    @pl.when(i == 0)
    def _per_code_tile():
        w_t = w_ref[...]
        c_scr[...] = jnp.sum(w_t * w_t, axis=1, keepdims=True)
        wwide_ref[:, 0:CODE_DIM] = w_t
        wwide_ref[:, CODE_DIM:2 * CODE_DIM] = w_t

    z = z2_ref[...]
    a = a_ref[...]
    c = c_scr[...]

    vals = [jnp.full((8, TT), jnp.inf, jnp.float32) for _ in range(_NACC)]
    pays = [jnp.zeros((8, TT), jnp.float32) for _ in range(_NACC)]
    for j in range(TK // _GRP):
        m2j = lax.dot_general(
            w_ref[j * _GRP:(j + 1) * _GRP, :], z, (((1,), (1,)), ((), ())),
            preferred_element_type=jnp.float32,
        )
        for gg in range(_GRP // 8):
            g = j * (_GRP // 8) + gg
            d = (a - m2j[gg * 8:(gg + 1) * 8, :]) + c[g * 8:(g + 1) * 8, :]
            pg = float(g)
            t = g % _NACC
            better = d < vals[t]
            vals[t] = jnp.minimum(d, vals[t])
            pays[t] = jnp.where(better, pg, pays[t])

    sub_iota = lax.broadcasted_iota(jnp.int32, (8, 1), 0).astype(jnp.float32)
    mval, midx = vals[0], pays[0] * 8.0 + sub_iota
    for t in range(1, _NACC):
        v2, i2 = vals[t], pays[t] * 8.0 + sub_iota
        b = (v2 < mval) | ((v2 == mval) & (i2 < midx))
        mval = jnp.where(b, v2, mval)
        midx = jnp.where(b, i2, midx)
    mcol = jnp.min(mval, axis=0, keepdims=True)
    sel = jnp.where(mval == mcol, midx, float(NUM_CODES))
    icol = jnp.min(sel, axis=0, keepdims=True)
    idx_ref[...] = icol.astype(jnp.int32)
    part = jnp.sum(mcol)
    prev = jnp.where(i == 0, 0.0, dsum_ref[0, 0])
    total = prev + part
    dsum_ref[0, 0] = jnp.where(i == _NI - 1, total * _LSCALE, total)


def _build_argmin(interpret: bool = False):
    return pl.pallas_call(
        _argmin_kernel,
        grid=(N_TOK // TT,),
        in_specs=[
            pl.BlockSpec((TK, CODE_DIM), lambda i: (0, 0)),
            pl.BlockSpec((TT, CODE_DIM), lambda i: (i, 0)),
            pl.BlockSpec((1, TT), lambda i: (0, i)),
        ],
        out_specs=[
            pl.BlockSpec((1, TT), lambda i: (0, i)),
            pl.BlockSpec((1, 1), lambda i: (0, 0),
                         memory_space=pltpu.SMEM),
            pl.BlockSpec((TK, 2 * CODE_DIM), lambda i: (0, 0)),
        ],
        out_shape=[
            jax.ShapeDtypeStruct((1, N_TOK), jnp.int32),
            jax.ShapeDtypeStruct((1, 1), jnp.float32),
            jax.ShapeDtypeStruct((NUM_CODES, 2 * CODE_DIM), jnp.float32),
        ],
        scratch_shapes=[
            pltpu.VMEM((TK, 1), jnp.float32),
        ],
        compiler_params=pltpu.CompilerParams(
            dimension_semantics=("arbitrary",),
        ),
        interpret=interpret,
    )


_NC = 2
_NS = 16
_NW = _NC * _NS
_TPW = N_TOK // _NW
_CHUNK = 96
_NCHUNK = _TPW // _CHUNK


def _sc_gather_body(w2_hbm, idx_hbm, z_hbm, out_hbm,
                    idx_v, rows_v, z_v, out_v, sem):
    wid = lax.axis_index("s") * _NC + lax.axis_index("c")
    base = wid * _TPW
    pltpu.sync_copy(idx_hbm.at[pl.ds(base, _TPW)], idx_v)
    cps = [
        pltpu.async_copy(
            w2_hbm.at[idx_v.at[pl.ds(c * _CHUNK, _CHUNK)]],
            rows_v.at[pl.ds(c * _CHUNK, _CHUNK)],
            sem,
        )
        for c in range(_NCHUNK)
    ]
    pltpu.sync_copy(z_hbm.at[pl.ds(base, _TPW)], z_v)
    for cp in cps:
        cp.wait()

    def body(r, carry):
        for j in range(CODE_DIM // 16):
            sl = pl.ds(j * 16, 16)
            q = rows_v[r, sl]
            zz = z_v[r, sl]
            out_v[r, sl] = zz + (q - zz)
        return carry

    lax.fori_loop(0, _TPW, body, 0)
    pltpu.sync_copy(out_v, out_hbm.at[pl.ds(base, _TPW)])


@functools.lru_cache(maxsize=1)
def _build_sc_gather():
    return pl.kernel(
        _sc_gather_body,
        mesh=plsc.VectorSubcoreMesh(core_axis_name="c", subcore_axis_name="s"),
        out_type=jax.ShapeDtypeStruct((N_TOK, CODE_DIM), jnp.float32),
        scratch_types=[
            pltpu.VMEM((_TPW,), jnp.int32),
            pltpu.VMEM((_TPW, 2 * CODE_DIM), jnp.float32),
            pltpu.VMEM((_TPW, CODE_DIM), jnp.float32),
            pltpu.VMEM((_TPW, CODE_DIM), jnp.float32),
            pltpu.SemaphoreType.DMA,
        ],
    )


def kernel(z, W):
    flat_z = z.reshape(-1, CODE_DIM)
    a = jnp.sum(flat_z ** 2, axis=1, keepdims=True).reshape(1, N_TOK)
    z2 = flat_z + flat_z
    idx2, dsum, w_wide = _build_argmin()(W, z2, a)
    idx_flat = idx2[0, :]
    z_q_ste = _build_sc_gather()(w_wide, idx_flat, flat_z).reshape(z.shape)
    return (z_q_ste, idx_flat.reshape(z.shape[:2]), dsum[0, 0])


# speedup vs baseline: 1.3053x; 1.0024x over previous
import functools

import jax
import jax.numpy as jnp
from jax import lax
from jax.experimental import pallas as pl
from jax.experimental.pallas import tpu as pltpu
from jax.experimental.pallas import tpu_sc as plsc

NUM_CODES = 8192
CODE_DIM = 64
COMMITMENT_COST = 0.25
N_TOK = 16 * 576

TT = 1024
TK = 8192
_NKS = NUM_CODES // TK
_NACC = 2


_GRP = 1024


_NI = N_TOK // TT
_LSCALE = COMMITMENT_COST / (N_TOK * CODE_DIM)


def _argmin_kernel(w_ref, z2_ref, a_ref, idx_ref, dsum_ref, wwide_ref,
                   c_scr):
    i = pl.program_id(0)

    @pl.when(i == 0)
    def _per_code_tile():
        w_t = w_ref[...]
        c_scr[...] = jnp.sum(w_t * w_t, axis=1, keepdims=True)
        wwide_ref[:, 0:CODE_DIM] = w_t
        wwide_ref[:, CODE_DIM:2 * CODE_DIM] = w_t

    z = z2_ref[...]
    a = a_ref[...]
    c = c_scr[...]

    vals = [jnp.full((8, TT), jnp.inf, jnp.float32) for _ in range(_NACC)]
    pays = [jnp.zeros((8, TT), jnp.float32) for _ in range(_NACC)]
    for j in range(TK // _GRP):
        m2j = lax.dot_general(
            w_ref[j * _GRP:(j + 1) * _GRP, :], z, (((1,), (1,)), ((), ())),
            preferred_element_type=jnp.float32,
        )
        for gg in range(_GRP // 8):
            g = j * (_GRP // 8) + gg
            d = (a - m2j[gg * 8:(gg + 1) * 8, :]) + c[g * 8:(g + 1) * 8, :]
            pg = float(g)
            t = g % _NACC
            better = d < vals[t]
            vals[t] = jnp.minimum(d, vals[t])
            pays[t] = jnp.where(better, pg, pays[t])

    sub_iota = lax.broadcasted_iota(jnp.int32, (8, 1), 0).astype(jnp.float32)
    mval, midx = vals[0], pays[0] * 8.0 + sub_iota
    for t in range(1, _NACC):
        v2, i2 = vals[t], pays[t] * 8.0 + sub_iota
        b = (v2 < mval) | ((v2 == mval) & (i2 < midx))
        mval = jnp.where(b, v2, mval)
        midx = jnp.where(b, i2, midx)
    mcol = jnp.min(mval, axis=0, keepdims=True)
    sel = jnp.where(mval == mcol, midx, float(NUM_CODES))
    icol = jnp.min(sel, axis=0, keepdims=True)
    idx_ref[...] = icol.astype(jnp.int32)
    part = jnp.sum(mcol)
    prev = jnp.where(i == 0, 0.0, dsum_ref[0, 0])
    total = prev + part
    dsum_ref[0, 0] = jnp.where(i == _NI - 1, total * _LSCALE, total)


def _build_argmin(interpret: bool = False):
    return pl.pallas_call(
        _argmin_kernel,
        grid=(N_TOK // TT,),
        in_specs=[
            pl.BlockSpec((TK, CODE_DIM), lambda i: (0, 0)),
            pl.BlockSpec((TT, CODE_DIM), lambda i: (i, 0)),
            pl.BlockSpec((1, TT), lambda i: (0, i)),
        ],
        out_specs=[
            pl.BlockSpec((1, TT), lambda i: (0, i)),
            pl.BlockSpec((1, 1), lambda i: (0, 0),
                         memory_space=pltpu.SMEM),
            pl.BlockSpec((TK, 2 * CODE_DIM), lambda i: (0, 0)),
        ],
        out_shape=[
            jax.ShapeDtypeStruct((1, N_TOK), jnp.int32),
            jax.ShapeDtypeStruct((1, 1), jnp.float32),
            jax.ShapeDtypeStruct((NUM_CODES, 2 * CODE_DIM), jnp.float32),
        ],
        scratch_shapes=[
            pltpu.VMEM((TK, 1), jnp.float32),
        ],
        compiler_params=pltpu.CompilerParams(
            dimension_semantics=("arbitrary",),
        ),
        interpret=interpret,
    )


_NC = 2
_NS = 16
_NW = _NC * _NS
_TPW = N_TOK // _NW
_CHUNK = 96
_NCHUNK = _TPW // _CHUNK


def _sc_gather_body(w2_hbm, idx_hbm, z_hbm, out_hbm,
                    idx_v, rows_v, z_v, out_v, sem):
    wid = lax.axis_index("s") * _NC + lax.axis_index("c")
    base = wid * _TPW
    pltpu.sync_copy(idx_hbm.at[pl.ds(base, _TPW)], idx_v)
    cps = [
        pltpu.async_copy(
            w2_hbm.at[idx_v.at[pl.ds(c * _CHUNK, _CHUNK)]],
            rows_v.at[pl.ds(c * _CHUNK, _CHUNK)],
            sem,
        )
        for c in range(_NCHUNK)
    ]
    pltpu.sync_copy(z_hbm.at[pl.ds(base, _TPW)], z_v)
    for cp in cps:
        cp.wait()

    def body(r, carry):
        for j in range(CODE_DIM // 16):
            sl = pl.ds(j * 16, 16)
            q = rows_v[r, sl]
            zz = z_v[r, sl]
            out_v[r, sl] = zz + (q - zz)
        return carry

    lax.fori_loop(0, _TPW, body, 0)
    pltpu.sync_copy(out_v, out_hbm.at[pl.ds(base, _TPW)])


@functools.lru_cache(maxsize=1)
def _build_sc_gather():
    return pl.kernel(
        _sc_gather_body,
        mesh=plsc.VectorSubcoreMesh(core_axis_name="c", subcore_axis_name="s"),
        out_type=jax.ShapeDtypeStruct((N_TOK, CODE_DIM), jnp.float32),
        scratch_types=[
            pltpu.VMEM((_TPW,), jnp.int32),
            pltpu.VMEM((_TPW, 2 * CODE_DIM), jnp.float32),
            pltpu.VMEM((_TPW, CODE_DIM), jnp.float32),
            pltpu.VMEM((_TPW, CODE_DIM), jnp.float32),
            pltpu.SemaphoreType.DMA,
        ],
    )


def kernel(z, W):
    flat_z = z.reshape(-1, CODE_DIM)
    a = jnp.sum(flat_z ** 2, axis=1, keepdims=True).reshape(1, N_TOK)
    z2 = flat_z + flat_z
    idx2, dsum, w_wide = _build_argmin()(W, z2, a)
    idx_flat = idx2[0, :]
    z_q_ste = _build_sc_gather()(w_wide, idx_flat, flat_z).reshape(z.shape)
    return (z_q_ste, idx_flat.reshape(z.shape[:2]), dsum[0, 0])
